# baseline (speedup 1.0000x reference)
import jax
import jax.numpy as jnp
from jax.experimental import pallas as pl
from jax.experimental.pallas import tpu as pltpu

_MEDNUM = 5000
_D = 128
_BM = 200


def _gcn_block(adj_ref, emb_ref, out_ref):
    h = jnp.dot(adj_ref[...], emb_ref[...], preferred_element_type=jnp.float32)
    out_ref[...] = 4.0 * jnp.maximum(h, 0.0)


def kernel(adj1, m1Embed, m2Embed):
    embeds = jnp.concatenate([m1Embed, m2Embed], axis=0)
    k = embeds.shape[0]
    return pl.pallas_call(
        _gcn_block,
        grid=(pl.cdiv(_MEDNUM, _BM),),
        in_specs=[
            pl.BlockSpec((_BM, k), lambda i: (i, 0)),
            pl.BlockSpec((k, _D), lambda i: (0, 0)),
        ],
        out_specs=pl.BlockSpec((_BM, _D), lambda i: (i, 0)),
        out_shape=jax.ShapeDtypeStruct((_MEDNUM, _D), jnp.float32),
        compiler_params=pltpu.CompilerParams(
            dimension_semantics=("parallel",),
        ),
    )(adj1, embeds)


# speedup vs baseline: 1.0668x; 1.0668x over previous
import jax
import jax.numpy as jnp
from jax.experimental import pallas as pl
from jax.experimental.pallas import tpu as pltpu

_MEDNUM = 5000
_D = 128
_BM = 200


def _gcn_block(adj_ref, m1_ref, m2_ref, out_ref, emb_ref):
    @pl.when(pl.program_id(0) == 0)
    def _():
        emb_ref[: _MEDNUM, :] = m1_ref[...]
        emb_ref[_MEDNUM :, :] = m2_ref[...]

    h = jnp.dot(adj_ref[...], emb_ref[...], preferred_element_type=jnp.float32)
    out_ref[...] = 4.0 * jnp.maximum(h, 0.0)


def kernel(adj1, m1Embed, m2Embed):
    k = 2 * _MEDNUM
    return pl.pallas_call(
        _gcn_block,
        grid=(pl.cdiv(_MEDNUM, _BM),),
        in_specs=[
            pl.BlockSpec((_BM, k), lambda i: (i, 0)),
            pl.BlockSpec((_MEDNUM, _D), lambda i: (0, 0)),
            pl.BlockSpec((_MEDNUM, _D), lambda i: (0, 0)),
        ],
        out_specs=pl.BlockSpec((_BM, _D), lambda i: (i, 0)),
        out_shape=jax.ShapeDtypeStruct((_MEDNUM, _D), jnp.float32),
        scratch_shapes=[pltpu.VMEM((k, _D), jnp.float32)],
    )(adj1, m1Embed, m2Embed)
